# baseline (speedup 1.0000x reference)
import functools

import jax
import jax.numpy as jnp
from jax import lax
from jax.experimental import pallas as pl
from jax.experimental.pallas import tpu as pltpu
from jax.experimental.pallas import tpu_sc as plsc

_BATCH = 4096
_DIM = 64
_BLK = 512
_NSTEP = _BATCH // _BLK
_EPS = 1e-12
_NUM_PAIRS = _BATCH * (_BATCH - 1) // 2

_NC = 2
_NS = 16
_NW = _NC * _NS
_BPW = _BATCH // _NW


_CH = 16


def _gather_body(uid_ref, pid_ref, utab_ref, itab_ref, uout_ref, pout_ref,
                 pair_v, out_v, idx_s, sem):
    wid = lax.axis_index("s") * _NC + lax.axis_index("c")
    base = wid * _BPW

    def one_table(id_hbm, tab_hbm, out_hbm):
        pltpu.sync_copy(id_hbm.at[pl.ds(base, _BPW)], idx_s)

        def select(coff):
            gp = idx_s[pl.ds(coff, _CH)]
            for r in range(_CH):
                hb = (gp[r] & 1) * _DIM
                for c in range(_DIM // 16):
                    out_v[coff + r, pl.ds(c * 16, 16)] = (
                        pair_v[coff + r, pl.ds(hb + c * 16, 16)])

        def chunk(ch, carry):
            off = ch * _CH
            gv = idx_s[pl.ds(off, _CH)]
            for r in range(_CH):
                line = gv[r] >> 1
                pltpu.make_async_copy(tab_hbm.at[pl.ds(line, 1)],
                                      pair_v.at[pl.ds(off + r, 1)], sem).start()

            @pl.when(ch > 0)
            def _drain_prev():
                pltpu.make_async_copy(tab_hbm.at[pl.ds(0, _CH)],
                                      pair_v.at[pl.ds(0, _CH)], sem).wait()
                select((ch - 1) * _CH)

            return carry

        lax.fori_loop(0, _BPW // _CH, chunk, 0)
        pltpu.make_async_copy(tab_hbm.at[pl.ds(0, _CH)],
                              pair_v.at[pl.ds(0, _CH)], sem).wait()
        select(_BPW - _CH)
        pltpu.sync_copy(out_v, out_hbm.at[pl.ds(base, _BPW)])

    one_table(uid_ref, utab_ref, uout_ref)
    one_table(pid_ref, itab_ref, pout_ref)


def _make_gather():
    return pl.kernel(
        _gather_body,
        mesh=plsc.VectorSubcoreMesh(core_axis_name="c", subcore_axis_name="s"),
        out_type=[jax.ShapeDtypeStruct((_BATCH, _DIM), jnp.float32)] * 2,
        scratch_types=[
            pltpu.VMEM((_BPW, 2 * _DIM), jnp.float32),
            pltpu.VMEM((_BPW, _DIM), jnp.float32),
            pltpu.VMEM((_BPW,), jnp.int32),
            pltpu.SemaphoreType.DMA,
        ],
    )


def _block_to_vreg(v):
    acc = lax.slice(v, (0, 0), (8, 128))
    for r in range(v.shape[0] // 8):
        for c in range(v.shape[1] // 128):
            if r == 0 and c == 0:
                continue
            acc = acc + lax.slice(v, (r * 8, c * 128), (r * 8 + 8, c * 128 + 128))
    return acc


def _loss_body(u_ref, p_ref, out_ref, un_ref, pn_ref,
               au_all, au_diag, ap_all, ap_diag, acc_ref, dma_sem):
    i = pl.program_id(0)
    j = pl.program_id(1)

    @pl.when((i == 0) & (j == 0))
    def _init():
        cu = pltpu.make_async_copy(u_ref, un_ref, dma_sem)
        cu.start()
        cu.wait()
        cp_ = pltpu.make_async_copy(p_ref, pn_ref, dma_sem)
        cp_.start()
        cp_.wait()
        u = un_ref[...]
        p = pn_ref[...]
        un = u / jnp.maximum(jnp.sqrt(jnp.sum(u * u, axis=1, keepdims=True)), _EPS)
        pn = p / jnp.maximum(jnp.sqrt(jnp.sum(p * p, axis=1, keepdims=True)), _EPS)
        un_ref[...] = un
        pn_ref[...] = pn
        d = un - pn
        acc_ref[0] = jnp.sum(d * d)
        ru = jnp.sum(un * un, axis=1, keepdims=True)
        rp = jnp.sum(pn * pn, axis=1, keepdims=True)
        acc_ref[1] = jnp.sum(jnp.exp(4.0 * ru - 4.0))
        acc_ref[2] = jnp.sum(jnp.exp(4.0 * rp - 4.0))
        zero = jnp.zeros((8, 128), jnp.float32)
        au_all[...] = zero
        au_diag[...] = zero
        ap_all[...] = zero
        ap_diag[...] = zero

    @pl.when(j >= i)
    def _compute():
        a_u = un_ref[pl.ds(i * _BLK, _BLK), :]
        b_u = un_ref[pl.ds(j * _BLK, _BLK), :]
        g_u = lax.dot_general(a_u, b_u, (((1,), (1,)), ((), ())),
                              preferred_element_type=jnp.float32)
        bv_u = _block_to_vreg(jnp.exp(4.0 * g_u - 4.0))
        au_all[...] += bv_u
        a_p = pn_ref[pl.ds(i * _BLK, _BLK), :]
        b_p = pn_ref[pl.ds(j * _BLK, _BLK), :]
        g_p = lax.dot_general(a_p, b_p, (((1,), (1,)), ((), ())),
                              preferred_element_type=jnp.float32)
        bv_p = _block_to_vreg(jnp.exp(4.0 * g_p - 4.0))
        ap_all[...] += bv_p

        @pl.when(i == j)
        def _diag():
            au_diag[...] += bv_u
            ap_diag[...] += bv_p

    @pl.when((i == _NSTEP - 1) & (j == _NSTEP - 1))
    def _fin():
        s_u = 2.0 * jnp.sum(au_all[...]) - jnp.sum(au_diag[...])
        s_p = 2.0 * jnp.sum(ap_all[...]) - jnp.sum(ap_diag[...])
        align = acc_ref[0] / _BATCH
        mean_u = (s_u - acc_ref[1]) * (0.5 / _NUM_PAIRS)
        mean_p = (s_p - acc_ref[2]) * (0.5 / _NUM_PAIRS)
        lu = jnp.log(jnp.full((1, 128), mean_u, jnp.float32))
        lp = jnp.log(jnp.full((1, 128), mean_p, jnp.float32))
        out_ref[...] = align + 0.5 * (lu + lp)


def _loss(u_emb, p_emb):
    out = pl.pallas_call(
        _loss_body,
        grid=(_NSTEP, _NSTEP),
        in_specs=[pl.BlockSpec(memory_space=pl.ANY)] * 2,
        out_specs=pl.BlockSpec((1, 128), lambda i, j: (0, 0)),
        out_shape=jax.ShapeDtypeStruct((1, 128), jnp.float32),
        scratch_shapes=[
            pltpu.VMEM((_BATCH, _DIM), jnp.float32),
            pltpu.VMEM((_BATCH, _DIM), jnp.float32),
            pltpu.VMEM((8, 128), jnp.float32),
            pltpu.VMEM((8, 128), jnp.float32),
            pltpu.VMEM((8, 128), jnp.float32),
            pltpu.VMEM((8, 128), jnp.float32),
            pltpu.SMEM((8,), jnp.float32),
            pltpu.SemaphoreType.DMA,
        ],
    )(u_emb, p_emb)
    return out[0, 0]


def kernel(user_id, pos_id, neg_id, user_table, item_table):
    ut2 = user_table.reshape(user_table.shape[0] // 2, 2 * _DIM)
    it2 = item_table.reshape(item_table.shape[0] // 2, 2 * _DIM)
    u_emb, p_emb = _make_gather()(user_id.astype(jnp.int32), pos_id.astype(jnp.int32),
                                  ut2, it2)
    return _loss(u_emb, p_emb)


# speedup vs baseline: 1.2827x; 1.2827x over previous
import jax
import jax.numpy as jnp
from jax import lax
from jax.experimental import pallas as pl
from jax.experimental.pallas import tpu as pltpu
from jax.experimental.pallas import tpu_sc as plsc

_BATCH = 4096
_DIM = 64
_BLK = 512
_NSTEP = _BATCH // _BLK
_EPS = 1e-12
_NUM_PAIRS = _BATCH * (_BATCH - 1) // 2

_NC = 2
_NS = 16
_NW = _NC * _NS
_BPW = _BATCH // _NW

_CH = 16


def _gather_body(id_ref, tab_ref, out_ref, out_v, idx_s, sem):
    wid = lax.axis_index("s") * _NC + lax.axis_index("c")
    base = wid * _BPW
    pltpu.sync_copy(id_ref.at[pl.ds(base, _BPW)], idx_s)

    def chunk(ch, carry):
        off = ch * _CH
        gv = idx_s[pl.ds(off, _CH)]
        for r in range(_CH):
            row = gv[r]
            pltpu.make_async_copy(tab_ref.at[pl.ds(row, 1)],
                                  out_v.at[pl.ds(off + r, 1)], sem).start()

        @pl.when(ch > 0)
        def _drain_prev():
            pltpu.make_async_copy(out_ref.at[pl.ds(base, _CH)],
                                  out_v.at[pl.ds(0, _CH)], sem).wait()

        return carry

    lax.fori_loop(0, _BPW // _CH, chunk, 0)
    pltpu.make_async_copy(out_ref.at[pl.ds(base, _CH)],
                          out_v.at[pl.ds(0, _CH)], sem).wait()
    pltpu.sync_copy(out_v, out_ref.at[pl.ds(base, _BPW)])


def _make_gather():
    return pl.kernel(
        _gather_body,
        mesh=plsc.VectorSubcoreMesh(core_axis_name="c", subcore_axis_name="s"),
        out_type=jax.ShapeDtypeStruct((_BATCH, _DIM), jnp.float32),
        scratch_types=[
            pltpu.VMEM((_BPW, _DIM), jnp.float32),
            pltpu.VMEM((_BPW,), jnp.int32),
            pltpu.SemaphoreType.DMA,
        ],
    )


def _block_to_vreg(v):
    acc = lax.slice(v, (0, 0), (8, 128))
    for r in range(v.shape[0] // 8):
        for c in range(v.shape[1] // 128):
            if r == 0 and c == 0:
                continue
            acc = acc + lax.slice(v, (r * 8, c * 128), (r * 8 + 8, c * 128 + 128))
    return acc


def _normalize_rows(x):
    return x / jnp.maximum(jnp.sqrt(jnp.sum(x * x, axis=1, keepdims=True)), _EPS)


def _gram_accumulate(n_ref, a_all, a_diag, i, j):
    a = n_ref[pl.ds(i * _BLK, _BLK), :]
    b = n_ref[pl.ds(j * _BLK, _BLK), :]
    g = lax.dot_general(a, b, (((1,), (1,)), ((), ())),
                        preferred_element_type=jnp.float32)
    bv = _block_to_vreg(jnp.exp(4.0 * g - 4.0))
    a_all[...] += bv

    @pl.when(i == j)
    def _diag():
        a_diag[...] += bv


def _loss_u_body(u_ref, un_out, su_out, du_out, un_ref, a_all, a_diag, acc_ref):
    i = pl.program_id(0)
    j = pl.program_id(1)

    @pl.when((i == 0) & (j == 0))
    def _init():
        un = _normalize_rows(u_ref[...])
        un_ref[...] = un
        un_out[...] = un
        ru = jnp.sum(un * un, axis=1, keepdims=True)
        acc_ref[0] = jnp.sum(jnp.exp(4.0 * ru - 4.0))
        zero = jnp.zeros((8, 128), jnp.float32)
        a_all[...] = zero
        a_diag[...] = zero

    @pl.when(j >= i)
    def _compute():
        _gram_accumulate(un_ref, a_all, a_diag, i, j)

    @pl.when((i == _NSTEP - 1) & (j == _NSTEP - 1))
    def _fin():
        s_u = 2.0 * jnp.sum(a_all[...]) - jnp.sum(a_diag[...])
        su_out[...] = jnp.full((1, 128), s_u, jnp.float32)
        du_out[...] = jnp.full((1, 128), acc_ref[0], jnp.float32)


def _loss_i_body(p_ref, un_ref_in, su_ref, du_ref, out_ref,
                 pn_ref, a_all, a_diag, acc_ref):
    i = pl.program_id(0)
    j = pl.program_id(1)

    @pl.when((i == 0) & (j == 0))
    def _init():
        pn = _normalize_rows(p_ref[...])
        pn_ref[...] = pn
        d = un_ref_in[...] - pn
        acc_ref[0] = jnp.sum(d * d)
        rp = jnp.sum(pn * pn, axis=1, keepdims=True)
        acc_ref[1] = jnp.sum(jnp.exp(4.0 * rp - 4.0))
        zero = jnp.zeros((8, 128), jnp.float32)
        a_all[...] = zero
        a_diag[...] = zero

    @pl.when(j >= i)
    def _compute():
        _gram_accumulate(pn_ref, a_all, a_diag, i, j)

    @pl.when((i == _NSTEP - 1) & (j == _NSTEP - 1))
    def _fin():
        s_p = 2.0 * jnp.sum(a_all[...]) - jnp.sum(a_diag[...])
        s_u = jnp.sum(su_ref[...]) * (1.0 / 128.0)
        d_u = jnp.sum(du_ref[...]) * (1.0 / 128.0)
        align = acc_ref[0] / _BATCH
        mean_u = (s_u - d_u) * (0.5 / _NUM_PAIRS)
        mean_p = (s_p - acc_ref[1]) * (0.5 / _NUM_PAIRS)
        lu = jnp.log(jnp.full((1, 128), mean_u, jnp.float32))
        lp = jnp.log(jnp.full((1, 128), mean_p, jnp.float32))
        out_ref[...] = align + 0.5 * (lu + lp)


def _loss_u(u_emb):
    return pl.pallas_call(
        _loss_u_body,
        grid=(_NSTEP, _NSTEP),
        in_specs=[pl.BlockSpec((_BATCH, _DIM), lambda i, j: (0, 0))],
        out_specs=[
            pl.BlockSpec((_BATCH, _DIM), lambda i, j: (0, 0)),
            pl.BlockSpec((1, 128), lambda i, j: (0, 0)),
            pl.BlockSpec((1, 128), lambda i, j: (0, 0)),
        ],
        out_shape=[
            jax.ShapeDtypeStruct((_BATCH, _DIM), jnp.float32),
            jax.ShapeDtypeStruct((1, 128), jnp.float32),
            jax.ShapeDtypeStruct((1, 128), jnp.float32),
        ],
        scratch_shapes=[
            pltpu.VMEM((_BATCH, _DIM), jnp.float32),
            pltpu.VMEM((8, 128), jnp.float32),
            pltpu.VMEM((8, 128), jnp.float32),
            pltpu.SMEM((8,), jnp.float32),
        ],
    )(u_emb)


def _loss_i(p_emb, un, su, du):
    out = pl.pallas_call(
        _loss_i_body,
        grid=(_NSTEP, _NSTEP),
        in_specs=[
            pl.BlockSpec((_BATCH, _DIM), lambda i, j: (0, 0)),
            pl.BlockSpec((_BATCH, _DIM), lambda i, j: (0, 0)),
            pl.BlockSpec((1, 128), lambda i, j: (0, 0)),
            pl.BlockSpec((1, 128), lambda i, j: (0, 0)),
        ],
        out_specs=pl.BlockSpec((1, 128), lambda i, j: (0, 0)),
        out_shape=jax.ShapeDtypeStruct((1, 128), jnp.float32),
        scratch_shapes=[
            pltpu.VMEM((_BATCH, _DIM), jnp.float32),
            pltpu.VMEM((8, 128), jnp.float32),
            pltpu.VMEM((8, 128), jnp.float32),
            pltpu.SMEM((8,), jnp.float32),
        ],
    )(p_emb, un, su, du)
    return out[0, 0]


def kernel(user_id, pos_id, neg_id, user_table, item_table):
    gather = _make_gather()
    u_emb = gather(user_id.astype(jnp.int32), user_table)
    p_emb = gather(pos_id.astype(jnp.int32), item_table)
    un, su, du = _loss_u(u_emb)
    return _loss_i(p_emb, un, su, du)


# speedup vs baseline: 1.3905x; 1.0840x over previous
import jax
import jax.numpy as jnp
from jax import lax
from jax.experimental import pallas as pl
from jax.experimental.pallas import tpu as pltpu
from jax.experimental.pallas import tpu_sc as plsc

_BATCH = 4096
_DIM = 64
_BLK = 512
_NSTEP = _BATCH // _BLK
_EPS = 1e-12
_NUM_PAIRS = _BATCH * (_BATCH - 1) // 2

_NC = 2
_NS = 16
_NW = _NC * _NS
_BPW = _BATCH // _NW

_CH = 16


def _gather_body(id_ref, tab_ref, out_ref, out_v, idx_s, sem):
    wid = lax.axis_index("s") * _NC + lax.axis_index("c")
    base = wid * _BPW
    pltpu.sync_copy(id_ref.at[pl.ds(base, _BPW)], idx_s)

    def chunk(ch, carry):
        off = ch * _CH
        gv = idx_s[pl.ds(off, _CH)]
        for r in range(_CH):
            row = gv[r]
            pltpu.make_async_copy(tab_ref.at[pl.ds(row, 1)],
                                  out_v.at[pl.ds(off + r, 1)], sem).start()

        @pl.when(ch > 0)
        def _drain_prev():
            pltpu.make_async_copy(out_ref.at[pl.ds(base, _CH)],
                                  out_v.at[pl.ds(0, _CH)], sem).wait()

        return carry

    lax.fori_loop(0, _BPW // _CH, chunk, 0)
    pltpu.make_async_copy(out_ref.at[pl.ds(base, _CH)],
                          out_v.at[pl.ds(0, _CH)], sem).wait()
    pltpu.sync_copy(out_v, out_ref.at[pl.ds(base, _BPW)])


def _make_gather():
    return pl.kernel(
        _gather_body,
        mesh=plsc.VectorSubcoreMesh(core_axis_name="c", subcore_axis_name="s"),
        out_type=jax.ShapeDtypeStruct((_BATCH, _DIM), jnp.float32),
        scratch_types=[
            pltpu.VMEM((_BPW, _DIM), jnp.float32),
            pltpu.VMEM((_BPW,), jnp.int32),
            pltpu.SemaphoreType.DMA,
        ],
    )


def _block_to_vreg(v):
    acc = lax.slice(v, (0, 0), (8, 128))
    for r in range(v.shape[0] // 8):
        for c in range(v.shape[1] // 128):
            if r == 0 and c == 0:
                continue
            acc = acc + lax.slice(v, (r * 8, c * 128), (r * 8 + 8, c * 128 + 128))
    return acc


def _loss_body(u_ref, p_ref, out_ref, un_ref, pn_ref,
               au_all, au_diag, ap_all, ap_diag, acc_ref, dma_sem):
    i = pl.program_id(0)
    j = pl.program_id(1)

    @pl.when((i == 0) & (j == 0))
    def _init():
        cu = pltpu.make_async_copy(u_ref, un_ref, dma_sem)
        cu.start()
        cu.wait()
        cp_ = pltpu.make_async_copy(p_ref, pn_ref, dma_sem)
        cp_.start()
        cp_.wait()
        u = un_ref[...]
        p = pn_ref[...]
        un = u / jnp.maximum(jnp.sqrt(jnp.sum(u * u, axis=1, keepdims=True)), _EPS)
        pn = p / jnp.maximum(jnp.sqrt(jnp.sum(p * p, axis=1, keepdims=True)), _EPS)
        un_ref[...] = un
        pn_ref[...] = pn
        d = un - pn
        acc_ref[0] = jnp.sum(d * d)
        ru = jnp.sum(un * un, axis=1, keepdims=True)
        rp = jnp.sum(pn * pn, axis=1, keepdims=True)
        acc_ref[1] = jnp.sum(jnp.exp(4.0 * ru - 4.0))
        acc_ref[2] = jnp.sum(jnp.exp(4.0 * rp - 4.0))
        zero = jnp.zeros((8, 128), jnp.float32)
        au_all[...] = zero
        au_diag[...] = zero
        ap_all[...] = zero
        ap_diag[...] = zero

    @pl.when(j >= i)
    def _compute():
        a_u = un_ref[pl.ds(i * _BLK, _BLK), :]
        b_u = un_ref[pl.ds(j * _BLK, _BLK), :]
        g_u = lax.dot_general(a_u, b_u, (((1,), (1,)), ((), ())),
                              preferred_element_type=jnp.float32)
        bv_u = _block_to_vreg(jnp.exp(4.0 * g_u - 4.0))
        au_all[...] += bv_u
        a_p = pn_ref[pl.ds(i * _BLK, _BLK), :]
        b_p = pn_ref[pl.ds(j * _BLK, _BLK), :]
        g_p = lax.dot_general(a_p, b_p, (((1,), (1,)), ((), ())),
                              preferred_element_type=jnp.float32)
        bv_p = _block_to_vreg(jnp.exp(4.0 * g_p - 4.0))
        ap_all[...] += bv_p

        @pl.when(i == j)
        def _diag():
            au_diag[...] += bv_u
            ap_diag[...] += bv_p

    @pl.when((i == _NSTEP - 1) & (j == _NSTEP - 1))
    def _fin():
        s_u = 2.0 * jnp.sum(au_all[...]) - jnp.sum(au_diag[...])
        s_p = 2.0 * jnp.sum(ap_all[...]) - jnp.sum(ap_diag[...])
        align = acc_ref[0] / _BATCH
        mean_u = (s_u - acc_ref[1]) * (0.5 / _NUM_PAIRS)
        mean_p = (s_p - acc_ref[2]) * (0.5 / _NUM_PAIRS)
        lu = jnp.log(jnp.full((1, 128), mean_u, jnp.float32))
        lp = jnp.log(jnp.full((1, 128), mean_p, jnp.float32))
        out_ref[...] = align + 0.5 * (lu + lp)


def _loss(u_emb, p_emb):
    out = pl.pallas_call(
        _loss_body,
        grid=(_NSTEP, _NSTEP),
        in_specs=[pl.BlockSpec(memory_space=pl.ANY)] * 2,
        out_specs=pl.BlockSpec((1, 128), lambda i, j: (0, 0)),
        out_shape=jax.ShapeDtypeStruct((1, 128), jnp.float32),
        scratch_shapes=[
            pltpu.VMEM((_BATCH, _DIM), jnp.float32),
            pltpu.VMEM((_BATCH, _DIM), jnp.float32),
            pltpu.VMEM((8, 128), jnp.float32),
            pltpu.VMEM((8, 128), jnp.float32),
            pltpu.VMEM((8, 128), jnp.float32),
            pltpu.VMEM((8, 128), jnp.float32),
            pltpu.SMEM((8,), jnp.float32),
            pltpu.SemaphoreType.DMA,
        ],
    )(u_emb, p_emb)
    return out[0, 0]


def kernel(user_id, pos_id, neg_id, user_table, item_table):
    gather = _make_gather()
    u_emb = gather(user_id.astype(jnp.int32), user_table)
    p_emb = gather(pos_id.astype(jnp.int32), item_table)
    return _loss(u_emb, p_emb)
